# baseline (speedup 1.0000x reference)
import functools

import jax
import jax.numpy as jnp
from jax import lax
from jax.experimental import pallas as pl
from jax.experimental.pallas import tpu as pltpu
from jax.experimental.pallas import tpu_sc as plsc

CHUNK = 128


@functools.cache
def _build_gather(n_total, emb):
    info = plsc.get_sparse_core_info()
    num_workers = info.num_cores * info.num_subcores
    assert n_total % (num_workers * CHUNK) == 0
    chunks_per_worker = n_total // (num_workers * CHUNK)
    per_worker = chunks_per_worker * CHUNK

    mesh = plsc.VectorSubcoreMesh(core_axis_name="c", subcore_axis_name="s")

    @functools.partial(
        pl.kernel,
        mesh=mesh,
        out_type=jax.ShapeDtypeStruct((n_total, emb), jnp.float32),
        scratch_types=[
            pltpu.VMEM((CHUNK,), jnp.int32),
            pltpu.VMEM((CHUNK, emb), jnp.float32),
            pltpu.SemaphoreType.DMA,
        ],
        compiler_params=pltpu.CompilerParams(use_tc_tiling_on_sc=False),
    )
    def gather(idx_hbm, table_hbm, out_hbm, idx_v, rows_v, sem):
        wid = lax.axis_index("s") * info.num_cores + lax.axis_index("c")
        base = wid * per_worker

        def body(g, carry):
            off = base + g * CHUNK
            pltpu.sync_copy(idx_hbm.at[pl.ds(off, CHUNK)], idx_v)
            pltpu.async_copy(table_hbm.at[idx_v], rows_v, sem).wait()
            pltpu.sync_copy(rows_v, out_hbm.at[pl.ds(off, CHUNK)])
            return carry

        lax.fori_loop(0, chunks_per_worker, body, 0)

    return gather


def kernel(x, table):
    b, l = x.shape
    _, emb = table.shape
    idx = x.reshape(b * l)
    out = _build_gather(b * l, emb)(idx, table)
    return out.reshape(b, l, emb)


# speedup vs baseline: 1.1916x; 1.1916x over previous
import functools

import jax
import jax.numpy as jnp
from jax import lax
from jax.experimental import pallas as pl
from jax.experimental.pallas import tpu as pltpu
from jax.experimental.pallas import tpu_sc as plsc

CHUNK = 128
K = 4
NBUF = 2


@functools.cache
def _build_gather(n_total, emb):
    info = plsc.get_sparse_core_info()
    num_workers = info.num_cores * info.num_subcores
    group = K * CHUNK
    assert n_total % (num_workers * group) == 0
    groups_per_worker = n_total // (num_workers * group)
    rows_per_worker = groups_per_worker * K

    mesh = plsc.VectorSubcoreMesh(core_axis_name="c", subcore_axis_name="s")

    @functools.partial(
        pl.kernel,
        mesh=mesh,
        out_type=jax.ShapeDtypeStruct((n_total, emb), jnp.float32),
        scratch_types=[
            pltpu.VMEM((NBUF, K, CHUNK), jnp.int32),
            pltpu.VMEM((NBUF, K * CHUNK, emb), jnp.float32),
            pltpu.SemaphoreType.DMA,
            pltpu.SemaphoreType.DMA,
            pltpu.SemaphoreType.DMA,
            pltpu.SemaphoreType.DMA,
            pltpu.SemaphoreType.DMA,
            pltpu.SemaphoreType.DMA,
        ],
        compiler_params=pltpu.CompilerParams(use_tc_tiling_on_sc=False),
    )
    def gather(idx_hbm, table_hbm, out_hbm, idx_v, rows_v,
               isem0, isem1, gsem0, gsem1, wsem0, wsem1):
        isem = [isem0, isem1]
        gsem = [gsem0, gsem1]
        wsem = [wsem0, wsem1]
        wid = lax.axis_index("s") * info.num_cores + lax.axis_index("c")
        row0 = wid * rows_per_worker

        def idx_load(p, s):
            return pltpu.make_async_copy(
                idx_hbm.at[pl.ds(row0 + p * K, K)], idx_v.at[s], isem[s])

        idx_load(0, 0).start()

        def body(i, carry):
            for s in range(NBUF):
                p = i * NBUF + s
                @pl.when(p >= NBUF)
                def _():
                    pltpu.make_async_copy(
                        rows_v.at[s],
                        out_hbm.at[pl.ds((row0 + (p - NBUF) * K) * CHUNK,
                                         K * CHUNK)],
                        wsem[s]).wait()

                idx_load(p, s).wait()
                gathers = [
                    pltpu.make_async_copy(
                        table_hbm.at[idx_v.at[s, j]],
                        rows_v.at[s, pl.ds(j * CHUNK, CHUNK)],
                        gsem[s])
                    for j in range(K)
                ]
                for g in gathers:
                    g.start()

                @pl.when(p + 1 < groups_per_worker)
                def _():
                    idx_load(p + 1, 1 - s).start()

                for g in gathers:
                    g.wait()
                pltpu.make_async_copy(
                    rows_v.at[s],
                    out_hbm.at[pl.ds((row0 + p * K) * CHUNK, K * CHUNK)],
                    wsem[s]).start()
            return carry

        lax.fori_loop(0, groups_per_worker // NBUF, body, 0)

        for s in range(NBUF):
            p = groups_per_worker - NBUF + s
            pltpu.make_async_copy(
                rows_v.at[s],
                out_hbm.at[pl.ds((row0 + p * K) * CHUNK, K * CHUNK)],
                wsem[s]).wait()

    return gather


def kernel(x, table):
    b, l = x.shape
    _, emb = table.shape
    n = b * l
    idx = x.reshape(n // CHUNK, CHUNK)
    out = _build_gather(n, emb)(idx, table)
    return out.reshape(b, l, emb)


# speedup vs baseline: 1.1974x; 1.0048x over previous
import functools

import jax
import jax.numpy as jnp
from jax import lax
from jax.experimental import pallas as pl
from jax.experimental.pallas import tpu as pltpu
from jax.experimental.pallas import tpu_sc as plsc

CHUNK = 128
K = 4
NBUF = 3


@functools.cache
def _build_gather(n_total, emb):
    info = plsc.get_sparse_core_info()
    num_workers = info.num_cores * info.num_subcores
    group = K * CHUNK
    assert n_total % (num_workers * group) == 0
    G = n_total // (num_workers * group)
    rows_per_worker = G * K
    assert G >= NBUF + 1

    mesh = plsc.VectorSubcoreMesh(core_axis_name="c", subcore_axis_name="s")

    @functools.partial(
        pl.kernel,
        mesh=mesh,
        out_type=jax.ShapeDtypeStruct((n_total, emb), jnp.float32),
        scratch_types=[
            pltpu.VMEM((NBUF, K, CHUNK), jnp.int32),
            pltpu.VMEM((NBUF, K * CHUNK, emb), jnp.float32),
        ]
        + [pltpu.SemaphoreType.DMA] * (3 * NBUF),
        compiler_params=pltpu.CompilerParams(use_tc_tiling_on_sc=False),
    )
    def gather(idx_hbm, table_hbm, out_hbm, idx_v, rows_v, *sems):
        isem = sems[0:NBUF]
        gsem = sems[NBUF:2 * NBUF]
        wsem = sems[2 * NBUF:3 * NBUF]
        wid = lax.axis_index("s") * info.num_cores + lax.axis_index("c")
        row0 = wid * rows_per_worker

        def idx_copy(p, s):
            return pltpu.make_async_copy(
                idx_hbm.at[pl.ds(row0 + p * K, K)], idx_v.at[s], isem[s])

        def gathers(p, s):
            return [
                pltpu.make_async_copy(
                    table_hbm.at[idx_v.at[s, j]],
                    rows_v.at[s, pl.ds(j * CHUNK, CHUNK)],
                    gsem[s])
                for j in range(K)
            ]

        def wr_copy(p, s):
            return pltpu.make_async_copy(
                rows_v.at[s],
                out_hbm.at[pl.ds((row0 + p * K) * CHUNK, K * CHUNK)],
                wsem[s])

        def fire(p, s, guard_rows):
            idx_copy(p, s).wait()
            if guard_rows:
                @pl.when(p >= NBUF)
                def _():
                    wr_copy(p - NBUF, s).wait()
            gs = gathers(p, s)
            for g in gs:
                g.start()
            return gs

        def drain(p, s):
            for g in gathers(p, s):
                g.wait()
            wr_copy(p, s).start()

        idx_copy(0, 0).start()
        fire(0, 0, guard_rows=False)
        idx_copy(1, 1).start()

        main_groups = G - 2
        iters = main_groups // NBUF

        def body(i, carry):
            for b in range(NBUF):
                p = 1 + i * NBUF + b
                s = (1 + b) % NBUF
                fire(p, s, guard_rows=True)
                idx_copy(p + 1, (s + 1) % NBUF).start()
                drain(p - 1, (s - 1) % NBUF)
            return carry

        lax.fori_loop(0, iters, body, 0)

        for p in range(1 + iters * NBUF, G):
            s = p % NBUF
            idx_copy(p, s).wait()
            if p >= NBUF:
                wr_copy(p - NBUF, s).wait()
            for g in gathers(p, s):
                g.start()
            if p + 1 < G:
                idx_copy(p + 1, (p + 1) % NBUF).start()
            drain(p - 1, (p - 1) % NBUF)

        drain(G - 1, (G - 1) % NBUF)
        for p in range(max(0, G - NBUF), G):
            wr_copy(p, p % NBUF).wait()

    return gather


def kernel(x, table):
    b, l = x.shape
    _, emb = table.shape
    n = b * l
    idx = x.reshape(n // CHUNK, CHUNK)
    out = _build_gather(n, emb)(idx, table)
    return out.reshape(b, l, emb)


# speedup vs baseline: 1.5912x; 1.3289x over previous
import functools

import jax
import jax.numpy as jnp
from jax import lax
from jax.experimental import pallas as pl
from jax.experimental.pallas import tpu as pltpu
from jax.experimental.pallas import tpu_sc as plsc

CHUNK = 128
K = 4
NBUF = 3


@functools.cache
def _build_gather(n_total, emb):
    info = plsc.get_sparse_core_info()
    num_workers = info.num_cores * info.num_subcores
    group = K * CHUNK
    assert n_total % (num_workers * group) == 0
    G = n_total // (num_workers * group)
    rows_per_worker = G * K
    assert G >= NBUF + 1

    mesh = plsc.VectorSubcoreMesh(core_axis_name="c", subcore_axis_name="s")

    @functools.partial(
        pl.kernel,
        mesh=mesh,
        out_type=jax.ShapeDtypeStruct((n_total, 2 * emb), jnp.float32),
        scratch_types=[
            pltpu.VMEM((NBUF, K, CHUNK), jnp.int32),
            pltpu.VMEM((NBUF, K * CHUNK, emb), jnp.float32),
        ]
        + [pltpu.SemaphoreType.DMA] * (3 * NBUF),
        compiler_params=pltpu.CompilerParams(use_tc_tiling_on_sc=False),
    )
    def gather(idx_hbm, table_hbm, out_hbm, idx_v, rows_v, *sems):
        isem = sems[0:NBUF]
        gsem = sems[NBUF:2 * NBUF]
        wsem = sems[2 * NBUF:3 * NBUF]
        wid = lax.axis_index("s") * info.num_cores + lax.axis_index("c")
        row0 = wid * rows_per_worker

        def idx_copy(p, s):
            return pltpu.make_async_copy(
                idx_hbm.at[pl.ds(row0 + p * K, K)], idx_v.at[s], isem[s])

        def gathers(p, s):
            return [
                pltpu.make_async_copy(
                    table_hbm.at[idx_v.at[s, j]],
                    rows_v.at[s, pl.ds(j * CHUNK, CHUNK)],
                    gsem[s])
                for j in range(K)
            ]

        def wr_copy(p, s):
            return pltpu.make_async_copy(
                rows_v.at[s],
                out_hbm.at[pl.ds((row0 + p * K) * CHUNK, K * CHUNK),
                           pl.ds(0, emb)],
                wsem[s])

        def fire(p, s, guard_rows):
            idx_copy(p, s).wait()
            if guard_rows:
                @pl.when(p >= NBUF)
                def _():
                    wr_copy(p - NBUF, s).wait()
            gs = gathers(p, s)
            for g in gs:
                g.start()
            return gs

        def drain(p, s):
            for g in gathers(p, s):
                g.wait()
            wr_copy(p, s).start()

        idx_copy(0, 0).start()
        fire(0, 0, guard_rows=False)
        idx_copy(1, 1).start()

        main_groups = G - 2
        iters = main_groups // NBUF

        def body(i, carry):
            for b in range(NBUF):
                p = 1 + i * NBUF + b
                s = (1 + b) % NBUF
                fire(p, s, guard_rows=True)
                idx_copy(p + 1, (s + 1) % NBUF).start()
                drain(p - 1, (s - 1) % NBUF)
            return carry

        lax.fori_loop(0, iters, body, 0)

        for p in range(1 + iters * NBUF, G):
            s = p % NBUF
            idx_copy(p, s).wait()
            if p >= NBUF:
                wr_copy(p - NBUF, s).wait()
            for g in gathers(p, s):
                g.start()
            if p + 1 < G:
                idx_copy(p + 1, (p + 1) % NBUF).start()
            drain(p - 1, (p - 1) % NBUF)

        drain(G - 1, (G - 1) % NBUF)
        for p in range(max(0, G - NBUF), G):
            wr_copy(p, p % NBUF).wait()

    return gather


def kernel(x, table):
    b, l = x.shape
    _, emb = table.shape
    n = b * l
    idx = x.reshape(n // CHUNK, CHUNK)
    out = _build_gather(n, emb)(idx, table)
    return out[:, :emb].reshape(b, l, emb)
